# baseline (speedup 1.0000x reference)
import functools

import jax
import jax.numpy as jnp
import numpy as np
from jax import lax
from jax.experimental import pallas as pl
from jax.experimental.pallas import tpu as pltpu
from jax.experimental.pallas import tpu_sc as plsc

D = 128
L = 16
NC = 2
NS = 16
NW = NC * NS
CHUNK = 64
ROWS_PER_TILE = 640
NPAD = NS * ROWS_PER_TILE

_SRC_OF = np.empty(D, np.int32)
for _m in range(D):
    _t, _r = divmod(_m, 2 * L)
    _SRC_OF[_m] = 2 * L * _t + _r // 2 + (L if (_r & 1) else 0)


def _proj_body(x_ref, wq_ref, bq_ref, wk_ref, bk_ref, wv_ref, bv_ref,
               ws_ref, bs_ref, q_ref, kv_ref, s_ref):
    xb = x_ref[...]
    q_ref[...] = jnp.dot(xb, wq_ref[...], preferred_element_type=jnp.float32) + bq_ref[...]
    kv_ref[:, :D] = (jnp.dot(xb, wk_ref[...], preferred_element_type=jnp.float32)
                     + bk_ref[...]).astype(jnp.bfloat16)
    kv_ref[:, D:] = (jnp.dot(xb, wv_ref[...], preferred_element_type=jnp.float32)
                     + bv_ref[...]).astype(jnp.bfloat16)
    s_ref[...] = jnp.dot(xb, ws_ref[...], preferred_element_type=jnp.float32) + bs_ref[...]


def _projections(x, Wq, bq, Wk, bk, Wv, bv, Ws, bs):
    n = x.shape[0]
    blk = 2000
    grid = (n // blk,)
    row_spec = pl.BlockSpec((blk, D), lambda i: (i, 0))
    kv_spec = pl.BlockSpec((blk, 2 * D), lambda i: (i, 0))
    full_spec = pl.BlockSpec((D, D), lambda i: (0, 0))
    bias_spec = pl.BlockSpec((1, D), lambda i: (0, 0))
    return pl.pallas_call(
        _proj_body,
        grid=grid,
        in_specs=[row_spec, full_spec, bias_spec, full_spec, bias_spec,
                  full_spec, bias_spec, full_spec, bias_spec],
        out_specs=[row_spec, kv_spec, row_spec],
        out_shape=[jax.ShapeDtypeStruct((n, D), jnp.float32),
                   jax.ShapeDtypeStruct((n, 2 * D), jnp.bfloat16),
                   jax.ShapeDtypeStruct((n, D), jnp.float32)],
    )(x, Wq, bq.reshape(1, D), Wk, bk.reshape(1, D),
      Wv, bv.reshape(1, D), Ws, bs.reshape(1, D))


def _edge_body(src_hbm, dst_hbm, q_hbm, kv_hbm, u_out, den_out,
               idx_s, idx_d, idx_dp, qd, kv, pbuf, den_local, u_sp, sem_i, sem_g):
    cid = lax.axis_index("c")
    sid = lax.axis_index("s")
    wid = sid * NC + cid
    n_edges = src_hbm.shape[0]
    n_chunks = n_edges // CHUNK
    chunks_per_tile = (n_chunks + NW - 1) // NW
    scale = 1.0 / (float(D) ** 0.5)
    zero16 = jnp.zeros((L,), jnp.float32)
    lane0 = lax.iota(jnp.int32, L) == 0
    fmt = plsc.PackFormat.INTERLEAVED

    def _zden(r, _):
        den_local[pl.ds(r * L, L)] = zero16
        return 0
    lax.fori_loop(0, NPAD // L + 1, _zden, 0)

    def _zrow(r, _):
        for j in range(D // L):
            qd[0, r, pl.ds(j * L, L)] = zero16
        return 0
    lax.fori_loop(0, CHUNK, _zrow, 0)
    for t in range(ROWS_PER_TILE // CHUNK):
        pltpu.sync_copy(qd.at[0],
                        u_sp.at[pl.ds(sid * ROWS_PER_TILE + t * CHUNK, CHUNK)])
    plsc.subcore_barrier()

    def _fire_idx(j, slot):
        c = j * NW + wid

        @pl.when(c < n_chunks)
        def _():
            base = c * CHUNK
            pltpu.async_copy(src_hbm.at[pl.ds(base, CHUNK)], idx_s.at[slot], sem_i)
            pltpu.async_copy(dst_hbm.at[pl.ds(base, CHUNK)], idx_d.at[slot], sem_i)
            pltpu.async_copy(dst_hbm.at[pl.ds(base, CHUNK)],
                             idx_dp.at[slot, pl.ds(0, CHUNK)], sem_i)

    def _wait_idx(j, slot):
        c = j * NW + wid

        @pl.when(c < n_chunks)
        def _():
            base = c * CHUNK
            pltpu.make_async_copy(src_hbm.at[pl.ds(base, CHUNK)], idx_s.at[slot], sem_i).wait()
            pltpu.make_async_copy(src_hbm.at[pl.ds(base, CHUNK)], idx_d.at[slot], sem_i).wait()
            pltpu.make_async_copy(src_hbm.at[pl.ds(base, CHUNK)],
                                  idx_dp.at[slot, pl.ds(0, CHUNK)], sem_i).wait()

    def _fire_gather(j, slot):
        c = j * NW + wid

        @pl.when(c < n_chunks)
        def _():
            pltpu.async_copy(q_hbm.at[idx_d.at[slot]], qd.at[slot], sem_g)
            pltpu.async_copy(kv_hbm.at[idx_s.at[slot]], kv.at[slot], sem_g)

    def _wait_gather(j, slot):
        c = j * NW + wid

        @pl.when(c < n_chunks)
        def _():
            pltpu.make_async_copy(q_hbm.at[idx_d.at[slot]], qd.at[slot], sem_g).wait()
            pltpu.make_async_copy(kv_hbm.at[idx_s.at[slot]], kv.at[slot], sem_g).wait()

    def _compute_scatter(j, slot):
        c = j * NW + wid

        @pl.when(c < n_chunks)
        def _():
            def _edge_logit(e, _):
                acca = zero16
                accb = zero16
                for t in range(D // (2 * L)):
                    kk = plsc.bitcast(kv[slot, e, pl.ds(t * L, L)], jnp.bfloat16)
                    ka, kb = plsc.unpack(kk, format=fmt)
                    acca = acca + qd[slot, e, pl.ds(2 * t * L, L)] * ka
                    accb = accb + qd[slot, e, pl.ds((2 * t + 1) * L, L)] * kb
                logit = jnp.sum(acca + accb, axis=0) * scale
                p = jnp.exp(jnp.full((L,), logit, jnp.float32))
                pbuf[pl.ds(e, L)] = p
                return 0
            lax.fori_loop(0, CHUNK, _edge_logit, 0)

            def _edge_scale(e, _):
                pv = pbuf[pl.ds(e, L)]
                d = idx_dp[slot, pl.ds(e, L)][0]
                cur = den_local[pl.ds(d, L)]
                den_local[pl.ds(d, L)] = cur + jnp.where(lane0, pv, 0.0)
                p = jnp.full((L,), pv[0], jnp.float32)
                for t in range(D // (2 * L)):
                    vv = plsc.bitcast(kv[slot, e, pl.ds((D // 2) + t * L, L)],
                                      jnp.bfloat16)
                    va, vb = plsc.unpack(vv, format=fmt)
                    qd[slot, e, pl.ds(2 * t * L, L)] = va * p
                    qd[slot, e, pl.ds((2 * t + 1) * L, L)] = vb * p
                return 0
            lax.fori_loop(0, CHUNK, _edge_scale, 0)
            pltpu.sync_copy(qd.at[slot], u_sp.at[idx_d.at[slot]], add=True)

    _fire_idx(0, 0)
    _wait_idx(0, 0)
    _fire_gather(0, 0)
    _fire_idx(1, 1)

    def _outer(o, _):
        for b in range(2):
            j = o * 2 + b
            _wait_gather(j, b)
            _wait_idx(j + 1, 1 - b)
            _fire_gather(j + 1, 1 - b)
            _compute_scatter(j, b)
            _fire_idx(j + 2, b)
        return 0

    lax.fori_loop(0, (chunks_per_tile + 1) // 2, _outer, 0)
    plsc.subcore_barrier()

    pltpu.sync_copy(u_sp.at[pl.ds(sid * ROWS_PER_TILE, ROWS_PER_TILE)],
                    u_out.at[cid, pl.ds(sid * ROWS_PER_TILE, ROWS_PER_TILE)])
    pltpu.sync_copy(den_local.at[pl.ds(0, NPAD)], den_out.at[wid])


def _edge_phase(src, dst, q, kv32):
    mesh = plsc.VectorSubcoreMesh(core_axis_name="c", subcore_axis_name="s")
    fn = pl.kernel(
        _edge_body,
        out_type=[jax.ShapeDtypeStruct((NC, NPAD, D), jnp.float32),
                  jax.ShapeDtypeStruct((NW, NPAD), jnp.float32)],
        mesh=mesh,
        compiler_params=pltpu.CompilerParams(needs_layout_passes=False),
        scratch_types=[
            pltpu.VMEM((2, CHUNK), jnp.int32),
            pltpu.VMEM((2, CHUNK), jnp.int32),
            pltpu.VMEM((2, CHUNK + L), jnp.int32),
            pltpu.VMEM((2, CHUNK, D), jnp.float32),
            pltpu.VMEM((2, CHUNK, D), jnp.int32),
            pltpu.VMEM((CHUNK + L,), jnp.float32),
            pltpu.VMEM((NPAD + L,), jnp.float32),
            pltpu.VMEM_SHARED((NPAD, D), jnp.float32),
            pltpu.SemaphoreType.DMA,
            pltpu.SemaphoreType.DMA,
        ],
    )
    return fn(src, dst, q, kv32)


def _combine_body(u_ref, den_ref, skip_ref, out_ref):
    ub = u_ref[0] + u_ref[1]
    den = jnp.sum(den_ref[...], axis=0)
    den = jnp.where(den == 0.0, 1.0, den)
    out_ref[...] = jnp.maximum(ub / den[:, None] + skip_ref[...], 0.0)


def _combine(u, den, skip):
    n = skip.shape[0]
    blk = 2048
    return pl.pallas_call(
        _combine_body,
        grid=(pl.cdiv(n, blk),),
        in_specs=[pl.BlockSpec((NC, blk, D), lambda i: (0, i, 0)),
                  pl.BlockSpec((NW, blk), lambda i: (0, i)),
                  pl.BlockSpec((blk, D), lambda i: (i, 0))],
        out_specs=pl.BlockSpec((blk, D), lambda i: (i, 0)),
        out_shape=jax.ShapeDtypeStruct((n, D), jnp.float32),
    )(u, den, skip)


def kernel(x, edge_index, Wq, bq, Wk, bk, Wv, bv, Ws, bs):
    src = edge_index[0]
    dst = edge_index[1]
    perm = jnp.asarray(_SRC_OF)
    q, kv_bf, skip = _projections(x, Wq, bq, Wk[:, perm], bk[perm],
                                  Wv[:, perm], bv[perm], Ws, bs)
    n = q.shape[0]
    kv32 = lax.bitcast_convert_type(kv_bf.reshape(n, D, 2), jnp.int32)
    u, den = _edge_phase(src, dst, q, kv32)
    return _combine(u, den, skip)


# speedup vs baseline: 1.3833x; 1.3833x over previous
import functools

import jax
import jax.numpy as jnp
import numpy as np
from jax import lax
from jax.experimental import pallas as pl
from jax.experimental.pallas import tpu as pltpu
from jax.experimental.pallas import tpu_sc as plsc

D = 128
L = 16
NC = 2
NS = 16
NW = NC * NS
CHUNK = 64
ROWS_PER_TILE = 640
NPAD = NS * ROWS_PER_TILE

_SRC_OF = np.empty(D, np.int32)
for _m in range(D):
    _t, _r = divmod(_m, 2 * L)
    _SRC_OF[_m] = 2 * L * _t + _r // 2 + (L if (_r & 1) else 0)


def _proj_body(x_ref, wq_ref, bq_ref, wk_ref, bk_ref, wv_ref, bv_ref,
               ws_ref, bs_ref, q_ref, kv_ref, s_ref):
    xb = x_ref[...]
    q_ref[...] = jnp.dot(xb, wq_ref[...], preferred_element_type=jnp.float32) + bq_ref[...]
    kv_ref[:, :D] = (jnp.dot(xb, wk_ref[...], preferred_element_type=jnp.float32)
                     + bk_ref[...]).astype(jnp.bfloat16)
    kv_ref[:, D:] = (jnp.dot(xb, wv_ref[...], preferred_element_type=jnp.float32)
                     + bv_ref[...]).astype(jnp.bfloat16)
    s_ref[...] = jnp.dot(xb, ws_ref[...], preferred_element_type=jnp.float32) + bs_ref[...]


def _projections(x, Wq, bq, Wk, bk, Wv, bv, Ws, bs):
    n = x.shape[0]
    blk = 2000
    grid = (n // blk,)
    row_spec = pl.BlockSpec((blk, D), lambda i: (i, 0))
    kv_spec = pl.BlockSpec((blk, 2 * D), lambda i: (i, 0))
    full_spec = pl.BlockSpec((D, D), lambda i: (0, 0))
    bias_spec = pl.BlockSpec((1, D), lambda i: (0, 0))
    return pl.pallas_call(
        _proj_body,
        grid=grid,
        in_specs=[row_spec, full_spec, bias_spec, full_spec, bias_spec,
                  full_spec, bias_spec, full_spec, bias_spec],
        out_specs=[row_spec, kv_spec, row_spec],
        out_shape=[jax.ShapeDtypeStruct((n, D), jnp.float32),
                   jax.ShapeDtypeStruct((n, 2 * D), jnp.bfloat16),
                   jax.ShapeDtypeStruct((n, D), jnp.float32)],
    )(x, Wq, bq.reshape(1, D), Wk, bk.reshape(1, D),
      Wv, bv.reshape(1, D), Ws, bs.reshape(1, D))


def _edge_body(src_hbm, dst_hbm, q_hbm, kv_hbm, u_out, den_out,
               idx_s, idx_d, idx_dp, qd, kv, pbuf, den_local, u_sp, sem_i, sem_g):
    cid = lax.axis_index("c")
    sid = lax.axis_index("s")
    wid = sid * NC + cid
    n_edges = src_hbm.shape[0]
    n_chunks = n_edges // CHUNK
    chunks_per_tile = (n_chunks + NW - 1) // NW
    scale = 1.0 / (float(D) ** 0.5)
    zero16 = jnp.zeros((L,), jnp.float32)
    lane0 = lax.iota(jnp.int32, L) == 0
    fmt = plsc.PackFormat.INTERLEAVED

    def _zden(r, _):
        den_local[pl.ds(r * L, L)] = zero16
        return 0
    lax.fori_loop(0, NPAD // L + 1, _zden, 0)

    def _zrow(r, _):
        for j in range(D // L):
            qd[0, r, pl.ds(j * L, L)] = zero16
        return 0
    lax.fori_loop(0, CHUNK, _zrow, 0)
    for t in range(ROWS_PER_TILE // CHUNK):
        pltpu.sync_copy(qd.at[0],
                        u_sp.at[pl.ds(sid * ROWS_PER_TILE + t * CHUNK, CHUNK)])
    plsc.subcore_barrier()

    def _fire_idx(j, slot):
        c = j * NW + wid

        @pl.when(c < n_chunks)
        def _():
            base = c * CHUNK
            pltpu.async_copy(src_hbm.at[pl.ds(base, CHUNK)], idx_s.at[slot], sem_i)
            pltpu.async_copy(dst_hbm.at[pl.ds(base, CHUNK)], idx_d.at[slot], sem_i)
            pltpu.async_copy(dst_hbm.at[pl.ds(base, CHUNK)],
                             idx_dp.at[slot, pl.ds(0, CHUNK)], sem_i)

    def _wait_idx(j, slot):
        c = j * NW + wid

        @pl.when(c < n_chunks)
        def _():
            base = c * CHUNK
            pltpu.make_async_copy(src_hbm.at[pl.ds(base, CHUNK)], idx_s.at[slot], sem_i).wait()
            pltpu.make_async_copy(src_hbm.at[pl.ds(base, CHUNK)], idx_d.at[slot], sem_i).wait()
            pltpu.make_async_copy(src_hbm.at[pl.ds(base, CHUNK)],
                                  idx_dp.at[slot, pl.ds(0, CHUNK)], sem_i).wait()

    def _fire_gather(j, slot):
        c = j * NW + wid

        @pl.when(c < n_chunks)
        def _():
            pltpu.async_copy(q_hbm.at[idx_d.at[slot]], qd.at[slot], sem_g)
            pltpu.async_copy(kv_hbm.at[idx_s.at[slot]], kv.at[slot], sem_g)

    def _wait_gather(j, slot):
        c = j * NW + wid

        @pl.when(c < n_chunks)
        def _():
            pltpu.make_async_copy(q_hbm.at[idx_d.at[slot]], qd.at[slot], sem_g).wait()
            pltpu.make_async_copy(kv_hbm.at[idx_s.at[slot]], kv.at[slot], sem_g).wait()

    def _compute_scatter(j, slot):
        c = j * NW + wid

        @pl.when(c < n_chunks)
        def _():
            def _edge_logit(e, _):
                acca = zero16
                accb = zero16
                for t in range(D // (2 * L)):
                    kk = plsc.bitcast(kv[slot, e, pl.ds(t * L, L)], jnp.bfloat16)
                    ka, kb = plsc.unpack(kk, format=fmt)
                    acca = acca + qd[slot, e, pl.ds(2 * t * L, L)] * ka
                    accb = accb + qd[slot, e, pl.ds((2 * t + 1) * L, L)] * kb
                logit = jnp.sum(acca + accb, axis=0) * scale
                p = jnp.exp(jnp.full((L,), logit, jnp.float32))
                pbuf[pl.ds(e, L)] = p
                return 0
            lax.fori_loop(0, CHUNK, _edge_logit, 0)

            for g in range(CHUNK // L):
                plsc.addupdate_scatter(
                    den_local, [idx_dp[slot, pl.ds(g * L, L)]],
                    pbuf[pl.ds(g * L, L)])

            def _edge_scale(e, _):
                p = jnp.full((L,), pbuf[pl.ds(e, L)][0], jnp.float32)
                for t in range(D // (2 * L)):
                    vv = plsc.bitcast(kv[slot, e, pl.ds((D // 2) + t * L, L)],
                                      jnp.bfloat16)
                    va, vb = plsc.unpack(vv, format=fmt)
                    qd[slot, e, pl.ds(2 * t * L, L)] = va * p
                    qd[slot, e, pl.ds((2 * t + 1) * L, L)] = vb * p
                return 0
            lax.fori_loop(0, CHUNK, _edge_scale, 0)
            pltpu.sync_copy(qd.at[slot], u_sp.at[idx_d.at[slot]], add=True)

    _fire_idx(0, 0)
    _wait_idx(0, 0)
    _fire_gather(0, 0)
    _fire_idx(1, 1)

    def _outer(o, _):
        for b in range(2):
            j = o * 2 + b
            _wait_gather(j, b)
            _wait_idx(j + 1, 1 - b)
            _fire_gather(j + 1, 1 - b)
            _compute_scatter(j, b)
            _fire_idx(j + 2, b)
        return 0

    lax.fori_loop(0, (chunks_per_tile + 1) // 2, _outer, 0)
    plsc.subcore_barrier()

    pltpu.sync_copy(u_sp.at[pl.ds(sid * ROWS_PER_TILE, ROWS_PER_TILE)],
                    u_out.at[cid, pl.ds(sid * ROWS_PER_TILE, ROWS_PER_TILE)])
    pltpu.sync_copy(den_local.at[pl.ds(0, NPAD)], den_out.at[wid])


def _edge_phase(src, dst, q, kv32):
    mesh = plsc.VectorSubcoreMesh(core_axis_name="c", subcore_axis_name="s")
    fn = pl.kernel(
        _edge_body,
        out_type=[jax.ShapeDtypeStruct((NC, NPAD, D), jnp.float32),
                  jax.ShapeDtypeStruct((NW, NPAD), jnp.float32)],
        mesh=mesh,
        compiler_params=pltpu.CompilerParams(needs_layout_passes=False),
        scratch_types=[
            pltpu.VMEM((2, CHUNK), jnp.int32),
            pltpu.VMEM((2, CHUNK), jnp.int32),
            pltpu.VMEM((2, CHUNK + L), jnp.int32),
            pltpu.VMEM((2, CHUNK, D), jnp.float32),
            pltpu.VMEM((2, CHUNK, D), jnp.int32),
            pltpu.VMEM((CHUNK + L,), jnp.float32),
            pltpu.VMEM((NPAD + L,), jnp.float32),
            pltpu.VMEM_SHARED((NPAD, D), jnp.float32),
            pltpu.SemaphoreType.DMA,
            pltpu.SemaphoreType.DMA,
        ],
    )
    return fn(src, dst, q, kv32)


def _combine_body(u_ref, den_ref, skip_ref, out_ref):
    ub = u_ref[0] + u_ref[1]
    den = jnp.sum(den_ref[...], axis=0)
    den = jnp.where(den == 0.0, 1.0, den)
    out_ref[...] = jnp.maximum(ub / den[:, None] + skip_ref[...], 0.0)


def _combine(u, den, skip):
    n = skip.shape[0]
    blk = 2048
    return pl.pallas_call(
        _combine_body,
        grid=(pl.cdiv(n, blk),),
        in_specs=[pl.BlockSpec((NC, blk, D), lambda i: (0, i, 0)),
                  pl.BlockSpec((NW, blk), lambda i: (0, i)),
                  pl.BlockSpec((blk, D), lambda i: (i, 0))],
        out_specs=pl.BlockSpec((blk, D), lambda i: (i, 0)),
        out_shape=jax.ShapeDtypeStruct((n, D), jnp.float32),
    )(u, den, skip)


def kernel(x, edge_index, Wq, bq, Wk, bk, Wv, bv, Ws, bs):
    src = edge_index[0]
    dst = edge_index[1]
    perm = jnp.asarray(_SRC_OF)
    q, kv_bf, skip = _projections(x, Wq, bq, Wk[:, perm], bk[perm],
                                  Wv[:, perm], bv[perm], Ws, bs)
    n = q.shape[0]
    kv32 = lax.bitcast_convert_type(kv_bf.reshape(n, D, 2), jnp.int32)
    u, den = _edge_phase(src, dst, q, kv32)
    return _combine(u, den, skip)


# speedup vs baseline: 1.4860x; 1.0742x over previous
import functools

import jax
import jax.numpy as jnp
import numpy as np
from jax import lax
from jax.experimental import pallas as pl
from jax.experimental.pallas import tpu as pltpu
from jax.experimental.pallas import tpu_sc as plsc

D = 128
L = 16
NC = 2
NS = 16
NW = NC * NS
CHUNK = 64
ROWS_PER_TILE = 640
NPAD = NS * ROWS_PER_TILE

_SRC_OF = np.empty(D, np.int32)
for _m in range(D):
    _t, _r = divmod(_m, 2 * L)
    _SRC_OF[_m] = 2 * L * _t + _r // 2 + (L if (_r & 1) else 0)


def _proj_body(x_ref, wq_ref, bq_ref, wk_ref, bk_ref, wv_ref, bv_ref,
               ws_ref, bs_ref, q_ref, kv_ref, s_ref):
    xb = x_ref[...]
    q_ref[...] = jnp.dot(xb, wq_ref[...], preferred_element_type=jnp.float32) + bq_ref[...]
    kv_ref[:, :D] = (jnp.dot(xb, wk_ref[...], preferred_element_type=jnp.float32)
                     + bk_ref[...]).astype(jnp.bfloat16)
    kv_ref[:, D:] = (jnp.dot(xb, wv_ref[...], preferred_element_type=jnp.float32)
                     + bv_ref[...]).astype(jnp.bfloat16)
    s_ref[...] = jnp.dot(xb, ws_ref[...], preferred_element_type=jnp.float32) + bs_ref[...]


def _projections(x, Wq, bq, Wk, bk, Wv, bv, Ws, bs):
    n = x.shape[0]
    blk = 2000
    grid = (n // blk,)
    row_spec = pl.BlockSpec((blk, D), lambda i: (i, 0))
    kv_spec = pl.BlockSpec((blk, 2 * D), lambda i: (i, 0))
    full_spec = pl.BlockSpec((D, D), lambda i: (0, 0))
    bias_spec = pl.BlockSpec((1, D), lambda i: (0, 0))
    return pl.pallas_call(
        _proj_body,
        grid=grid,
        in_specs=[row_spec, full_spec, bias_spec, full_spec, bias_spec,
                  full_spec, bias_spec, full_spec, bias_spec],
        out_specs=[row_spec, kv_spec, row_spec],
        out_shape=[jax.ShapeDtypeStruct((n, D), jnp.float32),
                   jax.ShapeDtypeStruct((n, 2 * D), jnp.bfloat16),
                   jax.ShapeDtypeStruct((n, D), jnp.float32)],
    )(x, Wq, bq.reshape(1, D), Wk, bk.reshape(1, D),
      Wv, bv.reshape(1, D), Ws, bs.reshape(1, D))


def _edge_body(src_hbm, dst_hbm, q_hbm, kv_hbm, u_out, den_out,
               idx_s, idx_d, idx_dp, idx_sc, qd, kv, pbuf, den_local, u_sp,
               sem_i, sem_g, sem_si, sem_s):
    cid = lax.axis_index("c")
    sid = lax.axis_index("s")
    wid = sid * NC + cid
    n_edges = src_hbm.shape[0]
    n_chunks = n_edges // CHUNK
    chunks_per_tile = (n_chunks + NW - 1) // NW
    scale = 1.0 / (float(D) ** 0.5)
    zero16 = jnp.zeros((L,), jnp.float32)
    lane0 = lax.iota(jnp.int32, L) == 0
    fmt = plsc.PackFormat.INTERLEAVED

    def _zden(r, _):
        den_local[pl.ds(r * L, L)] = zero16
        return 0
    lax.fori_loop(0, NPAD // L + 1, _zden, 0)

    def _zrow(r, _):
        for j in range(D // L):
            qd[0, r, pl.ds(j * L, L)] = zero16
        return 0
    lax.fori_loop(0, CHUNK, _zrow, 0)
    for t in range(ROWS_PER_TILE // CHUNK):
        pltpu.sync_copy(qd.at[0],
                        u_sp.at[pl.ds(sid * ROWS_PER_TILE + t * CHUNK, CHUNK)])
    plsc.subcore_barrier()

    def _fire_idx(j, slot):
        c = j * NW + wid

        @pl.when(c < n_chunks)
        def _():
            base = c * CHUNK
            pltpu.async_copy(src_hbm.at[pl.ds(base, CHUNK)], idx_s.at[slot], sem_i)
            pltpu.async_copy(dst_hbm.at[pl.ds(base, CHUNK)], idx_d.at[slot], sem_i)
            pltpu.async_copy(dst_hbm.at[pl.ds(base, CHUNK)],
                             idx_dp.at[slot, pl.ds(0, CHUNK)], sem_i)

    def _wait_idx(j, slot):
        c = j * NW + wid

        @pl.when(c < n_chunks)
        def _():
            base = c * CHUNK
            pltpu.make_async_copy(src_hbm.at[pl.ds(base, CHUNK)], idx_s.at[slot], sem_i).wait()
            pltpu.make_async_copy(src_hbm.at[pl.ds(base, CHUNK)], idx_d.at[slot], sem_i).wait()
            pltpu.make_async_copy(src_hbm.at[pl.ds(base, CHUNK)],
                                  idx_dp.at[slot, pl.ds(0, CHUNK)], sem_i).wait()

    def _fire_scidx(j, slot):
        c = j * NW + wid

        @pl.when(c < n_chunks)
        def _():
            base = c * CHUNK
            pltpu.async_copy(dst_hbm.at[pl.ds(base, CHUNK)], idx_sc.at[slot], sem_si)

    def _wait_scidx(j, slot):
        c = j * NW + wid

        @pl.when(c < n_chunks)
        def _():
            base = c * CHUNK
            pltpu.make_async_copy(dst_hbm.at[pl.ds(base, CHUNK)],
                                  idx_sc.at[slot], sem_si).wait()

    def _wait_scatter(j, slot):
        c = j * NW + wid

        @pl.when(jnp.logical_and(j >= 0, c < n_chunks))
        def _():
            pltpu.make_async_copy(qd.at[slot], u_sp.at[idx_sc.at[slot]], sem_s).wait()

    def _fire_gather(j, slot):
        c = j * NW + wid

        @pl.when(c < n_chunks)
        def _():
            pltpu.async_copy(q_hbm.at[idx_d.at[slot]], qd.at[slot], sem_g)
            pltpu.async_copy(kv_hbm.at[idx_s.at[slot]], kv.at[slot], sem_g)

    def _wait_gather(j, slot):
        c = j * NW + wid

        @pl.when(c < n_chunks)
        def _():
            pltpu.make_async_copy(q_hbm.at[idx_d.at[slot]], qd.at[slot], sem_g).wait()
            pltpu.make_async_copy(kv_hbm.at[idx_s.at[slot]], kv.at[slot], sem_g).wait()

    def _compute_scatter(j, slot):
        c = j * NW + wid

        @pl.when(c < n_chunks)
        def _():
            def _edge_logit(e, _):
                acca = zero16
                accb = zero16
                for t in range(D // (2 * L)):
                    kk = plsc.bitcast(kv[slot, e, pl.ds(t * L, L)], jnp.bfloat16)
                    ka, kb = plsc.unpack(kk, format=fmt)
                    acca = acca + qd[slot, e, pl.ds(2 * t * L, L)] * ka
                    accb = accb + qd[slot, e, pl.ds((2 * t + 1) * L, L)] * kb
                logit = jnp.sum(acca + accb, axis=0) * scale
                p = jnp.exp(jnp.full((L,), logit, jnp.float32))
                pbuf[pl.ds(e, L)] = p
                return 0
            lax.fori_loop(0, CHUNK, _edge_logit, 0)

            for g in range(CHUNK // L):
                plsc.addupdate_scatter(
                    den_local, [idx_dp[slot, pl.ds(g * L, L)]],
                    pbuf[pl.ds(g * L, L)])

            def _edge_scale(e, _):
                p = jnp.full((L,), pbuf[pl.ds(e, L)][0], jnp.float32)
                for t in range(D // (2 * L)):
                    vv = plsc.bitcast(kv[slot, e, pl.ds((D // 2) + t * L, L)],
                                      jnp.bfloat16)
                    va, vb = plsc.unpack(vv, format=fmt)
                    qd[slot, e, pl.ds(2 * t * L, L)] = va * p
                    qd[slot, e, pl.ds((2 * t + 1) * L, L)] = vb * p
                return 0
            lax.fori_loop(0, CHUNK, _edge_scale, 0)
            _wait_scidx(j, slot)
            pltpu.async_copy(qd.at[slot], u_sp.at[idx_sc.at[slot]], sem_s, add=True)

    _fire_idx(0, 0)
    _wait_idx(0, 0)
    _fire_scidx(0, 0)
    _fire_gather(0, 0)
    _fire_idx(1, 1)

    def _outer(o, _):
        for b in range(2):
            j = o * 2 + b
            _wait_gather(j, b)
            _wait_idx(j + 1, 1 - b)
            _wait_scatter(j - 1, 1 - b)
            _fire_scidx(j + 1, 1 - b)
            _fire_gather(j + 1, 1 - b)
            _compute_scatter(j, b)
            _fire_idx(j + 2, b)
        return 0

    lax.fori_loop(0, (chunks_per_tile + 1) // 2, _outer, 0)
    if chunks_per_tile % 2 == 0:
        _wait_scatter(chunks_per_tile - 1, (chunks_per_tile - 1) % 2)
    plsc.subcore_barrier()

    pltpu.sync_copy(u_sp.at[pl.ds(sid * ROWS_PER_TILE, ROWS_PER_TILE)],
                    u_out.at[cid, pl.ds(sid * ROWS_PER_TILE, ROWS_PER_TILE)])
    pltpu.sync_copy(den_local.at[pl.ds(0, NPAD)], den_out.at[wid])


def _edge_phase(src, dst, q, kv32):
    mesh = plsc.VectorSubcoreMesh(core_axis_name="c", subcore_axis_name="s")
    fn = pl.kernel(
        _edge_body,
        out_type=[jax.ShapeDtypeStruct((NC, NPAD, D), jnp.float32),
                  jax.ShapeDtypeStruct((NW, NPAD), jnp.float32)],
        mesh=mesh,
        compiler_params=pltpu.CompilerParams(needs_layout_passes=False),
        scratch_types=[
            pltpu.VMEM((2, CHUNK), jnp.int32),
            pltpu.VMEM((2, CHUNK), jnp.int32),
            pltpu.VMEM((2, CHUNK + L), jnp.int32),
            pltpu.VMEM((2, CHUNK), jnp.int32),
            pltpu.VMEM((2, CHUNK, D), jnp.float32),
            pltpu.VMEM((2, CHUNK, D), jnp.int32),
            pltpu.VMEM((CHUNK + L,), jnp.float32),
            pltpu.VMEM((NPAD + L,), jnp.float32),
            pltpu.VMEM_SHARED((NPAD, D), jnp.float32),
            pltpu.SemaphoreType.DMA,
            pltpu.SemaphoreType.DMA,
            pltpu.SemaphoreType.DMA,
            pltpu.SemaphoreType.DMA,
        ],
    )
    return fn(src, dst, q, kv32)


def _combine_body(u_ref, den_ref, skip_ref, out_ref):
    ub = u_ref[0] + u_ref[1]
    den = jnp.sum(den_ref[...], axis=0)
    den = jnp.where(den == 0.0, 1.0, den)
    out_ref[...] = jnp.maximum(ub / den[:, None] + skip_ref[...], 0.0)


def _combine(u, den, skip):
    n = skip.shape[0]
    blk = 2048
    return pl.pallas_call(
        _combine_body,
        grid=(pl.cdiv(n, blk),),
        in_specs=[pl.BlockSpec((NC, blk, D), lambda i: (0, i, 0)),
                  pl.BlockSpec((NW, blk), lambda i: (0, i)),
                  pl.BlockSpec((blk, D), lambda i: (i, 0))],
        out_specs=pl.BlockSpec((blk, D), lambda i: (i, 0)),
        out_shape=jax.ShapeDtypeStruct((n, D), jnp.float32),
    )(u, den, skip)


def kernel(x, edge_index, Wq, bq, Wk, bk, Wv, bv, Ws, bs):
    src = edge_index[0]
    dst = edge_index[1]
    perm = jnp.asarray(_SRC_OF)
    q, kv_bf, skip = _projections(x, Wq, bq, Wk[:, perm], bk[perm],
                                  Wv[:, perm], bv[perm], Ws, bs)
    n = q.shape[0]
    kv32 = lax.bitcast_convert_type(kv_bf.reshape(n, D, 2), jnp.int32)
    u, den = _edge_phase(src, dst, q, kv32)
    return _combine(u, den, skip)


# speedup vs baseline: 1.9780x; 1.3311x over previous
import functools

import jax
import jax.numpy as jnp
import numpy as np
from jax import lax
from jax.experimental import pallas as pl
from jax.experimental.pallas import tpu as pltpu
from jax.experimental.pallas import tpu_sc as plsc

D = 128
L = 16
NC = 2
NS = 16
NW = NC * NS
CHUNK = 64
ROWS_PER_TILE = 640
NPAD = NS * ROWS_PER_TILE

_SRC_OF = np.empty(D, np.int32)
for _m in range(D):
    _t, _r = divmod(_m, 2 * L)
    _SRC_OF[_m] = 2 * L * _t + _r // 2 + (L if (_r & 1) else 0)


def _proj_body(x_ref, wq_ref, bq_ref, wk_ref, bk_ref, wv_ref, bv_ref,
               ws_ref, bs_ref, q_ref, kv_ref, s_ref):
    xb = x_ref[...]
    q_ref[...] = jnp.dot(xb, wq_ref[...], preferred_element_type=jnp.float32) + bq_ref[...]
    kv_ref[:, :D] = (jnp.dot(xb, wk_ref[...], preferred_element_type=jnp.float32)
                     + bk_ref[...]).astype(jnp.bfloat16)
    kv_ref[:, D:] = (jnp.dot(xb, wv_ref[...], preferred_element_type=jnp.float32)
                     + bv_ref[...]).astype(jnp.bfloat16)
    s_ref[...] = jnp.dot(xb, ws_ref[...], preferred_element_type=jnp.float32) + bs_ref[...]


def _projections(x, Wq, bq, Wk, bk, Wv, bv, Ws, bs):
    n = x.shape[0]
    blk = 2000
    grid = (n // blk,)
    row_spec = pl.BlockSpec((blk, D), lambda i: (i, 0))
    kv_spec = pl.BlockSpec((blk, 2 * D), lambda i: (i, 0))
    full_spec = pl.BlockSpec((D, D), lambda i: (0, 0))
    bias_spec = pl.BlockSpec((1, D), lambda i: (0, 0))
    return pl.pallas_call(
        _proj_body,
        grid=grid,
        in_specs=[row_spec, full_spec, bias_spec, full_spec, bias_spec,
                  full_spec, bias_spec, full_spec, bias_spec],
        out_specs=[row_spec, kv_spec, row_spec],
        out_shape=[jax.ShapeDtypeStruct((n, D), jnp.float32),
                   jax.ShapeDtypeStruct((n, 2 * D), jnp.bfloat16),
                   jax.ShapeDtypeStruct((n, D), jnp.float32)],
    )(x, Wq, bq.reshape(1, D), Wk, bk.reshape(1, D),
      Wv, bv.reshape(1, D), Ws, bs.reshape(1, D))


def _edge_body(src_hbm, dst_hbm, q_hbm, kv_hbm, u_out, den_out,
               idx_s, idx_d, idx_sc, qd, kv, pbuf, den_local, u_sp,
               sem_i, sem_g, sem_si, sem_s):
    cid = lax.axis_index("c")
    sid = lax.axis_index("s")
    wid = sid * NC + cid
    n_edges = src_hbm.shape[0]
    n_chunks = n_edges // CHUNK
    chunks_per_tile = (n_chunks + NW - 1) // NW
    scale = 1.0 / (float(D) ** 0.5)
    zero16 = jnp.zeros((L,), jnp.float32)
    lane0 = lax.iota(jnp.int32, L) == 0
    fmt = plsc.PackFormat.INTERLEAVED

    def _zden(r, _):
        den_local[pl.ds(r * L, L)] = zero16
        return 0
    lax.fori_loop(0, NPAD // L + 1, _zden, 0)

    def _zrow(r, _):
        for j in range(D // L):
            qd[0, r, pl.ds(j * L, L)] = zero16
        return 0
    lax.fori_loop(0, CHUNK, _zrow, 0)
    for t in range(ROWS_PER_TILE // CHUNK):
        pltpu.sync_copy(qd.at[0],
                        u_sp.at[pl.ds(sid * ROWS_PER_TILE + t * CHUNK, CHUNK)])
    plsc.subcore_barrier()

    def _fire_idx(j, slot):
        c = j * NW + wid

        @pl.when(c < n_chunks)
        def _():
            base = c * CHUNK
            pltpu.async_copy(src_hbm.at[pl.ds(base, CHUNK)], idx_s.at[slot], sem_i)
            pltpu.async_copy(dst_hbm.at[pl.ds(base, CHUNK)], idx_d.at[slot], sem_i)

    def _wait_idx(j, slot):
        c = j * NW + wid

        @pl.when(c < n_chunks)
        def _():
            base = c * CHUNK
            pltpu.make_async_copy(src_hbm.at[pl.ds(base, CHUNK)], idx_s.at[slot], sem_i).wait()
            pltpu.make_async_copy(src_hbm.at[pl.ds(base, CHUNK)], idx_d.at[slot], sem_i).wait()

    def _fire_scidx(j, slot):
        c = j * NW + wid

        @pl.when(c < n_chunks)
        def _():
            base = c * CHUNK
            pltpu.async_copy(dst_hbm.at[pl.ds(base, CHUNK)], idx_sc.at[slot], sem_si)

    def _wait_scidx(j, slot):
        c = j * NW + wid

        @pl.when(c < n_chunks)
        def _():
            base = c * CHUNK
            pltpu.make_async_copy(dst_hbm.at[pl.ds(base, CHUNK)],
                                  idx_sc.at[slot], sem_si).wait()

    def _wait_scatter(j, slot):
        c = j * NW + wid

        @pl.when(jnp.logical_and(j >= 0, c < n_chunks))
        def _():
            pltpu.make_async_copy(qd.at[slot], u_sp.at[idx_sc.at[slot]], sem_s).wait()

    def _fire_gather(j, slot):
        c = j * NW + wid

        @pl.when(c < n_chunks)
        def _():
            pltpu.async_copy(q_hbm.at[idx_d.at[slot]], qd.at[slot], sem_g)
            pltpu.async_copy(kv_hbm.at[idx_s.at[slot]], kv.at[slot], sem_g)

    def _wait_gather(j, slot):
        c = j * NW + wid

        @pl.when(c < n_chunks)
        def _():
            pltpu.make_async_copy(q_hbm.at[idx_d.at[slot]], qd.at[slot], sem_g).wait()
            pltpu.make_async_copy(kv_hbm.at[idx_s.at[slot]], kv.at[slot], sem_g).wait()

    def _compute_scatter(j, slot):
        c = j * NW + wid

        @pl.when(c < n_chunks)
        def _():
            def _edge_logit(e, _):
                acca = zero16
                accb = zero16
                for t in range(D // (2 * L)):
                    kk = plsc.bitcast(kv[slot, e, pl.ds(t * L, L)], jnp.bfloat16)
                    ka, kb = plsc.unpack(kk, format=fmt)
                    acca = acca + qd[slot, e, pl.ds(2 * t * L, L)] * ka
                    accb = accb + qd[slot, e, pl.ds((2 * t + 1) * L, L)] * kb
                logit = jnp.sum(acca + accb, axis=0)
                pbuf[pl.ds(e, L)] = jnp.full((L,), logit, jnp.float32)
                return 0
            lax.fori_loop(0, CHUNK, _edge_logit, 0)

            for g in range(CHUNK // L):
                pbuf[pl.ds(g * L, L)] = jnp.exp(pbuf[pl.ds(g * L, L)] * scale)
            for g in range(CHUNK // L):
                plsc.addupdate_scatter(
                    den_local, [idx_d[slot, pl.ds(g * L, L)]],
                    pbuf[pl.ds(g * L, L)])

            def _edge_scale(e, _):
                p = jnp.full((L,), pbuf[pl.ds(e, L)][0], jnp.float32)
                for t in range(D // (2 * L)):
                    vv = plsc.bitcast(kv[slot, e, pl.ds((D // 2) + t * L, L)],
                                      jnp.bfloat16)
                    va, vb = plsc.unpack(vv, format=fmt)
                    qd[slot, e, pl.ds(2 * t * L, L)] = va * p
                    qd[slot, e, pl.ds((2 * t + 1) * L, L)] = vb * p
                return 0
            lax.fori_loop(0, CHUNK, _edge_scale, 0)
            _wait_scidx(j, slot)
            pltpu.async_copy(qd.at[slot], u_sp.at[idx_sc.at[slot]], sem_s, add=True)

    _fire_idx(0, 0)
    _wait_idx(0, 0)
    _fire_scidx(0, 0)
    _fire_gather(0, 0)
    _fire_idx(1, 1)

    def _outer(o, _):
        for b in range(2):
            j = o * 2 + b
            _wait_gather(j, b)
            _wait_idx(j + 1, 1 - b)
            _wait_scatter(j - 1, 1 - b)
            _fire_scidx(j + 1, 1 - b)
            _fire_gather(j + 1, 1 - b)
            _compute_scatter(j, b)
            _fire_idx(j + 2, b)
        return 0

    lax.fori_loop(0, (chunks_per_tile + 1) // 2, _outer, 0)
    if chunks_per_tile % 2 == 0:
        _wait_scatter(chunks_per_tile - 1, (chunks_per_tile - 1) % 2)
    plsc.subcore_barrier()

    pltpu.sync_copy(u_sp.at[pl.ds(sid * ROWS_PER_TILE, ROWS_PER_TILE)],
                    u_out.at[cid, pl.ds(sid * ROWS_PER_TILE, ROWS_PER_TILE)])
    pltpu.sync_copy(den_local.at[pl.ds(0, NPAD)], den_out.at[wid])


def _edge_phase(src, dst, q, kv32):
    mesh = plsc.VectorSubcoreMesh(core_axis_name="c", subcore_axis_name="s")
    fn = pl.kernel(
        _edge_body,
        out_type=[jax.ShapeDtypeStruct((NC, NPAD, D), jnp.float32),
                  jax.ShapeDtypeStruct((NW, NPAD), jnp.float32)],
        mesh=mesh,
        compiler_params=pltpu.CompilerParams(needs_layout_passes=False),
        scratch_types=[
            pltpu.VMEM((2, CHUNK), jnp.int32),
            pltpu.VMEM((2, CHUNK), jnp.int32),
            pltpu.VMEM((2, CHUNK), jnp.int32),
            pltpu.VMEM((2, CHUNK, D), jnp.float32),
            pltpu.VMEM((2, CHUNK, D), jnp.int32),
            pltpu.VMEM((CHUNK + L,), jnp.float32),
            pltpu.VMEM((NPAD + L,), jnp.float32),
            pltpu.VMEM_SHARED((NPAD, D), jnp.float32),
            pltpu.SemaphoreType.DMA,
            pltpu.SemaphoreType.DMA,
            pltpu.SemaphoreType.DMA,
            pltpu.SemaphoreType.DMA,
        ],
    )
    return fn(src, dst, q, kv32)


def _combine_body(u_ref, den_ref, skip_ref, out_ref):
    ub = u_ref[0] + u_ref[1]
    den = jnp.sum(den_ref[...], axis=0)
    den = jnp.where(den == 0.0, 1.0, den)
    out_ref[...] = jnp.maximum(ub / den[:, None] + skip_ref[...], 0.0)


def _combine(u, den, skip):
    n = skip.shape[0]
    blk = 2048
    return pl.pallas_call(
        _combine_body,
        grid=(pl.cdiv(n, blk),),
        in_specs=[pl.BlockSpec((NC, blk, D), lambda i: (0, i, 0)),
                  pl.BlockSpec((NW, blk), lambda i: (0, i)),
                  pl.BlockSpec((blk, D), lambda i: (i, 0))],
        out_specs=pl.BlockSpec((blk, D), lambda i: (i, 0)),
        out_shape=jax.ShapeDtypeStruct((n, D), jnp.float32),
    )(u, den, skip)


def kernel(x, edge_index, Wq, bq, Wk, bk, Wv, bv, Ws, bs):
    src = edge_index[0]
    dst = edge_index[1]
    perm = jnp.asarray(_SRC_OF)
    q, kv_bf, skip = _projections(x, Wq, bq, Wk[:, perm], bk[perm],
                                  Wv[:, perm], bv[perm], Ws, bs)
    n = q.shape[0]
    kv32 = lax.bitcast_convert_type(kv_bf.reshape(n, D, 2), jnp.int32)
    u, den = _edge_phase(src, dst, q, kv32)
    return _combine(u, den, skip)
